# baseline (speedup 1.0000x reference)
import functools

import jax
import jax.numpy as jnp
from jax import lax
from jax.experimental import pallas as pl
from jax.experimental.pallas import tpu as pltpu
from jax.experimental.pallas import tpu_sc as plsc

N = 10000
E = 320000
C = 128

NC = 2
NS = 16
NW = NC * NS
KCH = 128
NBUF = 2
IDXG = 8
NCH_W = -(-E // (KCH * NW * IDXG)) * IDXG
E_PAD = NCH_W * NW * KCH
NWIN = NCH_W // IDXG
RPT = 624
TAIL_OFF = RPT * NS
TAIL = N - TAIL_OFF

def _lane_splat(w16, e2):
    return lax.gather(
        w16, jnp.full((16, 1), e2, jnp.int32),
        lax.GatherDimensionNumbers(
            offset_dims=(), collapsed_slice_dims=(0,),
            start_index_map=(0,)),
        (1,), mode=lax.GatherScatterMode.PROMISE_IN_BOUNDS)


def _segsum_body(h_hbm, edge_hbm, ew_hbm, zeros_hbm, out_hbm,
                 src_a, dst_a, ew_a, rows_v, aggr_sh, gsems, ssems):
    cid = lax.axis_index("c")
    sid = lax.axis_index("s")
    wid = sid * NC + cid

    pltpu.sync_copy(zeros_hbm.at[pl.ds(sid * RPT, RPT)],
                    aggr_sh.at[pl.ds(sid * RPT, RPT)])

    @pl.when(sid == NS - 1)
    def _zero_tail():
        pltpu.sync_copy(zeros_hbm.at[pl.ds(TAIL_OFF, TAIL)],
                        aggr_sh.at[pl.ds(TAIL_OFF, TAIL)])

    plsc.subcore_barrier()

    def mult_chunk(c, b):
        def grp_body(g, c2):
            w16 = ew_a[c, pl.ds(g * 16, 16)]
            for e2 in range(16):
                wv = _lane_splat(w16, e2)
                e = g * 16 + e2
                for j in range(8):
                    rows_v[b, e, pl.ds(j * 16, 16)] = (
                        rows_v[b, e, pl.ds(j * 16, 16)] * wv)
            return c2

        lax.fori_loop(0, KCH // 16, grp_body, 0)

    def drain_scatter(b):
        pltpu.make_async_copy(h_hbm.at[pl.ds(0, KCH)], rows_v.at[b],
                              ssems.at[b]).wait()

    def win_body(s, carry):
        wbase = wid * NCH_W + s * IDXG
        pltpu.sync_copy(edge_hbm.at[0, pl.ds(wbase, IDXG), :], src_a)
        pltpu.sync_copy(edge_hbm.at[1, pl.ds(wbase, IDXG), :], dst_a)
        pltpu.sync_copy(ew_hbm.at[pl.ds(wbase, IDXG), :], ew_a)

        def group_body(g, c2):
            gds = []
            for b in range(NBUF):
                gds.append(pltpu.async_copy(
                    h_hbm.at[src_a.at[g * NBUF + b]], rows_v.at[b],
                    gsems.at[b]))
            for b in range(NBUF):
                c = g * NBUF + b
                gds[b].wait()
                mult_chunk(c, b)
                pltpu.sync_copy(rows_v.at[b], aggr_sh.at[dst_a.at[c]],
                                add=True)
            return c2

        lax.fori_loop(0, IDXG // NBUF, group_body, 0)
        return carry

    lax.fori_loop(0, NWIN, win_body, 0)
    plsc.subcore_barrier()
    pltpu.sync_copy(aggr_sh.at[pl.ds(sid * RPT, RPT)],
                    out_hbm.at[cid, pl.ds(sid * RPT, RPT)])

    @pl.when(sid == NS - 1)
    def _write_tail():
        pltpu.sync_copy(aggr_sh.at[pl.ds(TAIL_OFF, TAIL)],
                        out_hbm.at[cid, pl.ds(TAIL_OFF, TAIL)])


@functools.cache
def _segsum():
    mesh = plsc.VectorSubcoreMesh(core_axis_name="c", subcore_axis_name="s",
                                  num_cores=NC, num_subcores=NS)
    return pl.kernel(
        _segsum_body,
        out_type=jax.ShapeDtypeStruct((NC, N, C), jnp.float32),
        mesh=mesh,
        scratch_types=[
            pltpu.VMEM((IDXG, KCH), jnp.int32),
            pltpu.VMEM((IDXG, KCH), jnp.int32),
            pltpu.VMEM((IDXG, KCH), jnp.float32),
            pltpu.VMEM((NBUF, KCH, C), jnp.float32),
            pltpu.VMEM_SHARED((N, C), jnp.float32),
            pltpu.SemaphoreType.DMA((NBUF,)),
            pltpu.SemaphoreType.DMA((NBUF,)),
        ],
    )


BLK = 2000


def _tc_in_body(x_ref, wt_ref, b_ref, o_ref):
    y = jnp.dot(x_ref[...], wt_ref[...], preferred_element_type=jnp.float32)
    o_ref[...] = jnp.maximum(y + b_ref[...], 0.0)


def _tc_comb_body(p_ref, h_ref, wrelt_ref, wroott_ref, b_ref, g_ref, be_ref,
                  o_ref):
    aggr = p_ref[0] + p_ref[1]
    t = (jnp.dot(aggr, wrelt_ref[...], preferred_element_type=jnp.float32)
         + jnp.dot(h_ref[...], wroott_ref[...],
                   preferred_element_type=jnp.float32)
         + b_ref[...])
    m = jnp.mean(t, axis=-1, keepdims=True)
    v = jnp.mean((t - m) * (t - m), axis=-1, keepdims=True)
    t = (t - m) * lax.rsqrt(v + 1e-5) * g_ref[...] + be_ref[...]
    o_ref[...] = jnp.maximum(t, 0.0)


def _tc_out_body(p_ref, x1_ref, h_ref, wrelt_ref, wroott_ref, b_ref, g_ref,
                 be_ref, woutt_ref, bout_ref, x2_ref, out_ref):
    aggr = p_ref[0] + p_ref[1]
    t = (jnp.dot(aggr, wrelt_ref[...], preferred_element_type=jnp.float32)
         + jnp.dot(x1_ref[...], wroott_ref[...],
                   preferred_element_type=jnp.float32)
         + b_ref[...])
    m = jnp.mean(t, axis=-1, keepdims=True)
    v = jnp.mean((t - m) * (t - m), axis=-1, keepdims=True)
    t = (t - m) * lax.rsqrt(v + 1e-5) * g_ref[...] + be_ref[...]
    x2 = jnp.maximum(t + h_ref[...], 0.0)
    x2_ref[...] = x2
    out_ref[...] = (jnp.dot(x2, woutt_ref[...],
                            preferred_element_type=jnp.float32)
                    + bout_ref[...])


def _row_spec(blk):
    return pl.BlockSpec((blk, C), lambda i: (i, 0))


_W_SPEC = pl.BlockSpec((C, C), lambda i: (0, 0))
_V_SPEC = pl.BlockSpec((1, C), lambda i: (0, 0))
_P_SPEC = pl.BlockSpec((NC, BLK, C), lambda i: (0, i, 0))

_tc_in = pl.pallas_call(
    _tc_in_body,
    grid=(N // BLK,),
    in_specs=[_row_spec(BLK), _W_SPEC, _V_SPEC],
    out_specs=_row_spec(BLK),
    out_shape=jax.ShapeDtypeStruct((N, C), jnp.float32),
)

_tc_comb = pl.pallas_call(
    _tc_comb_body,
    grid=(N // BLK,),
    in_specs=[_P_SPEC, _row_spec(BLK), _W_SPEC, _W_SPEC, _V_SPEC, _V_SPEC,
              _V_SPEC],
    out_specs=_row_spec(BLK),
    out_shape=jax.ShapeDtypeStruct((N, C), jnp.float32),
)

_tc_out = pl.pallas_call(
    _tc_out_body,
    grid=(N // BLK,),
    in_specs=[_P_SPEC, _row_spec(BLK), _row_spec(BLK), _W_SPEC, _W_SPEC,
              _V_SPEC, _V_SPEC, _V_SPEC, _W_SPEC, _V_SPEC],
    out_specs=[_row_spec(BLK), _row_spec(BLK)],
    out_shape=[jax.ShapeDtypeStruct((N, C), jnp.float32),
               jax.ShapeDtypeStruct((N, C), jnp.float32)],
)


def kernel(x, edge, edgeweight, W_in, b_in, Wrel0, brel0, Wroot0, g0, be0,
           Wrel1, brel1, Wroot1, g1, be1, W_out, b_out):
    zeros = jnp.zeros((N, C), jnp.float32)
    edge3 = jnp.pad(edge, ((0, 0), (0, E_PAD - E))).reshape(2, -1, KCH)
    ew2 = jnp.pad(edgeweight, (0, E_PAD - E)).reshape(-1, KCH)
    h = _tc_in(x, W_in.T, b_in.reshape(1, C))
    p0 = _segsum()(h, edge3, ew2, zeros)
    x1 = _tc_comb(p0, h, Wrel0.T, Wroot0.T, brel0.reshape(1, C),
                  g0.reshape(1, C), be0.reshape(1, C))
    p1 = _segsum()(x1, edge3, ew2, zeros)
    x2, out = _tc_out(p1, x1, h, Wrel1.T, Wroot1.T, brel1.reshape(1, C),
                      g1.reshape(1, C), be1.reshape(1, C), W_out.T,
                      b_out.reshape(1, C))
    return (x2, out)


# speedup vs baseline: 1.2953x; 1.2953x over previous
import functools

import jax
import jax.numpy as jnp
from jax import lax
from jax.experimental import pallas as pl
from jax.experimental.pallas import tpu as pltpu
from jax.experimental.pallas import tpu_sc as plsc

N = 10000
E = 320000
C = 128

NC = 2
NS = 16
NW = NC * NS
KCH = 128
NBUF = 2
IDXG = 8
NCH_W = -(-E // (KCH * NW * IDXG)) * IDXG
E_PAD = NCH_W * NW * KCH
NWIN = NCH_W // IDXG
RPT = 624
TAIL_OFF = RPT * NS
TAIL = N - TAIL_OFF

def _lane_splat(w16, e2):
    return lax.gather(
        w16, jnp.full((16, 1), e2, jnp.int32),
        lax.GatherDimensionNumbers(
            offset_dims=(), collapsed_slice_dims=(0,),
            start_index_map=(0,)),
        (1,), mode=lax.GatherScatterMode.PROMISE_IN_BOUNDS)


def _segsum_body(h_hbm, edge_hbm, ew_hbm, zeros_hbm, out_hbm,
                 src_a, dst_a, ew_a, rows_v, aggr_sh, gsems, ssems):
    cid = lax.axis_index("c")
    sid = lax.axis_index("s")
    wid = sid * NC + cid

    pltpu.sync_copy(zeros_hbm.at[pl.ds(sid * RPT, RPT)],
                    aggr_sh.at[pl.ds(sid * RPT, RPT)])

    @pl.when(sid == NS - 1)
    def _zero_tail():
        pltpu.sync_copy(zeros_hbm.at[pl.ds(TAIL_OFF, TAIL)],
                        aggr_sh.at[pl.ds(TAIL_OFF, TAIL)])

    plsc.subcore_barrier()

    def mult_chunk(c, b):
        def grp_body(g, c2):
            w16 = ew_a[c, pl.ds(g * 16, 16)]
            for e2 in range(16):
                wv = _lane_splat(w16, e2)
                e = g * 16 + e2
                for j in range(8):
                    rows_v[b, e, pl.ds(j * 16, 16)] = (
                        rows_v[b, e, pl.ds(j * 16, 16)] * wv)
            return c2

        lax.fori_loop(0, KCH // 16, grp_body, 0)

    def drain_scatter(b):
        pltpu.make_async_copy(h_hbm.at[pl.ds(0, KCH)], rows_v.at[b],
                              ssems.at[b]).wait()

    def win_body(s, carry):
        wbase = wid * NCH_W + s * IDXG
        pltpu.sync_copy(edge_hbm.at[0, pl.ds(wbase, IDXG), :], src_a)
        pltpu.sync_copy(edge_hbm.at[1, pl.ds(wbase, IDXG), :], dst_a)
        pltpu.sync_copy(ew_hbm.at[pl.ds(wbase, IDXG), :], ew_a)

        def group_body(g, c2):
            gds = []
            for b in range(NBUF):
                @pl.when(g > 0)
                def _wait_prev():
                    drain_scatter(b)

                gds.append(pltpu.async_copy(
                    h_hbm.at[src_a.at[g * NBUF + b]], rows_v.at[b],
                    gsems.at[b]))
            for b in range(NBUF):
                c = g * NBUF + b
                gds[b].wait()
                mult_chunk(c, b)
                pltpu.async_copy(rows_v.at[b], aggr_sh.at[dst_a.at[c]],
                                 ssems.at[b], add=True)
            return c2

        lax.fori_loop(0, IDXG // NBUF, group_body, 0)
        for b in range(NBUF):
            drain_scatter(b)
        return carry

    lax.fori_loop(0, NWIN, win_body, 0)
    plsc.subcore_barrier()
    pltpu.sync_copy(aggr_sh.at[pl.ds(sid * RPT, RPT)],
                    out_hbm.at[cid, pl.ds(sid * RPT, RPT)])

    @pl.when(sid == NS - 1)
    def _write_tail():
        pltpu.sync_copy(aggr_sh.at[pl.ds(TAIL_OFF, TAIL)],
                        out_hbm.at[cid, pl.ds(TAIL_OFF, TAIL)])


@functools.cache
def _segsum():
    mesh = plsc.VectorSubcoreMesh(core_axis_name="c", subcore_axis_name="s",
                                  num_cores=NC, num_subcores=NS)
    return pl.kernel(
        _segsum_body,
        out_type=jax.ShapeDtypeStruct((NC, N, C), jnp.float32),
        mesh=mesh,
        scratch_types=[
            pltpu.VMEM((IDXG, KCH), jnp.int32),
            pltpu.VMEM((IDXG, KCH), jnp.int32),
            pltpu.VMEM((IDXG, KCH), jnp.float32),
            pltpu.VMEM((NBUF, KCH, C), jnp.float32),
            pltpu.VMEM_SHARED((N, C), jnp.float32),
            pltpu.SemaphoreType.DMA((NBUF,)),
            pltpu.SemaphoreType.DMA((NBUF,)),
        ],
    )


BLK = 2000


def _tc_in_body(x_ref, wt_ref, b_ref, o_ref):
    y = jnp.dot(x_ref[...], wt_ref[...], preferred_element_type=jnp.float32)
    o_ref[...] = jnp.maximum(y + b_ref[...], 0.0)


def _tc_comb_body(p_ref, h_ref, wrelt_ref, wroott_ref, b_ref, g_ref, be_ref,
                  o_ref):
    aggr = p_ref[0] + p_ref[1]
    t = (jnp.dot(aggr, wrelt_ref[...], preferred_element_type=jnp.float32)
         + jnp.dot(h_ref[...], wroott_ref[...],
                   preferred_element_type=jnp.float32)
         + b_ref[...])
    m = jnp.mean(t, axis=-1, keepdims=True)
    v = jnp.mean((t - m) * (t - m), axis=-1, keepdims=True)
    t = (t - m) * lax.rsqrt(v + 1e-5) * g_ref[...] + be_ref[...]
    o_ref[...] = jnp.maximum(t, 0.0)


def _tc_out_body(p_ref, x1_ref, h_ref, wrelt_ref, wroott_ref, b_ref, g_ref,
                 be_ref, woutt_ref, bout_ref, x2_ref, out_ref):
    aggr = p_ref[0] + p_ref[1]
    t = (jnp.dot(aggr, wrelt_ref[...], preferred_element_type=jnp.float32)
         + jnp.dot(x1_ref[...], wroott_ref[...],
                   preferred_element_type=jnp.float32)
         + b_ref[...])
    m = jnp.mean(t, axis=-1, keepdims=True)
    v = jnp.mean((t - m) * (t - m), axis=-1, keepdims=True)
    t = (t - m) * lax.rsqrt(v + 1e-5) * g_ref[...] + be_ref[...]
    x2 = jnp.maximum(t + h_ref[...], 0.0)
    x2_ref[...] = x2
    out_ref[...] = (jnp.dot(x2, woutt_ref[...],
                            preferred_element_type=jnp.float32)
                    + bout_ref[...])


def _row_spec(blk):
    return pl.BlockSpec((blk, C), lambda i: (i, 0))


_W_SPEC = pl.BlockSpec((C, C), lambda i: (0, 0))
_V_SPEC = pl.BlockSpec((1, C), lambda i: (0, 0))
_P_SPEC = pl.BlockSpec((NC, BLK, C), lambda i: (0, i, 0))

_tc_in = pl.pallas_call(
    _tc_in_body,
    grid=(N // BLK,),
    in_specs=[_row_spec(BLK), _W_SPEC, _V_SPEC],
    out_specs=_row_spec(BLK),
    out_shape=jax.ShapeDtypeStruct((N, C), jnp.float32),
)

_tc_comb = pl.pallas_call(
    _tc_comb_body,
    grid=(N // BLK,),
    in_specs=[_P_SPEC, _row_spec(BLK), _W_SPEC, _W_SPEC, _V_SPEC, _V_SPEC,
              _V_SPEC],
    out_specs=_row_spec(BLK),
    out_shape=jax.ShapeDtypeStruct((N, C), jnp.float32),
)

_tc_out = pl.pallas_call(
    _tc_out_body,
    grid=(N // BLK,),
    in_specs=[_P_SPEC, _row_spec(BLK), _row_spec(BLK), _W_SPEC, _W_SPEC,
              _V_SPEC, _V_SPEC, _V_SPEC, _W_SPEC, _V_SPEC],
    out_specs=[_row_spec(BLK), _row_spec(BLK)],
    out_shape=[jax.ShapeDtypeStruct((N, C), jnp.float32),
               jax.ShapeDtypeStruct((N, C), jnp.float32)],
)


def kernel(x, edge, edgeweight, W_in, b_in, Wrel0, brel0, Wroot0, g0, be0,
           Wrel1, brel1, Wroot1, g1, be1, W_out, b_out):
    zeros = jnp.zeros((N, C), jnp.float32)
    pad_n = E_PAD - E
    pad = jnp.stack([jnp.zeros((pad_n,), jnp.int32),
                     jnp.arange(pad_n, dtype=jnp.int32) % N])
    edge3 = jnp.concatenate([edge, pad], axis=1).reshape(2, -1, KCH)
    ew2 = jnp.pad(edgeweight, (0, pad_n)).reshape(-1, KCH)
    h = _tc_in(x, W_in.T, b_in.reshape(1, C))
    p0 = _segsum()(h, edge3, ew2, zeros)
    x1 = _tc_comb(p0, h, Wrel0.T, Wroot0.T, brel0.reshape(1, C),
                  g0.reshape(1, C), be0.reshape(1, C))
    p1 = _segsum()(x1, edge3, ew2, zeros)
    x2, out = _tc_out(p1, x1, h, Wrel1.T, Wroot1.T, brel1.reshape(1, C),
                      g1.reshape(1, C), be1.reshape(1, C), W_out.T,
                      b_out.reshape(1, C))
    return (x2, out)


# speedup vs baseline: 3.0268x; 2.3367x over previous
import functools

import jax
import jax.numpy as jnp
from jax import lax
from jax.experimental import pallas as pl
from jax.experimental.pallas import tpu as pltpu
from jax.experimental.pallas import tpu_sc as plsc

N = 10000
E = 320000
C = 128

NC = 2
NS = 16
NW = NC * NS
KCH = 128
NBUF = 2
IDXG = 8
NCH_W = -(-E // (KCH * NW * IDXG)) * IDXG
E_PAD = NCH_W * NW * KCH
NWIN = NCH_W // IDXG
RPT = 624
TAIL_OFF = RPT * NS
TAIL = N - TAIL_OFF

def _lane_splat(w16, e2):
    return lax.gather(
        w16, jnp.full((16, 1), e2, jnp.int32),
        lax.GatherDimensionNumbers(
            offset_dims=(), collapsed_slice_dims=(0,),
            start_index_map=(0,)),
        (1,), mode=lax.GatherScatterMode.PROMISE_IN_BOUNDS)


def _segsum_body(h_hbm, edge_hbm, ew_hbm, zeros_hbm, out_hbm,
                 src_a, dst_a, ew_a, rows_v, aggr_sh, gsems, ssems):
    cid = lax.axis_index("c")
    sid = lax.axis_index("s")
    wid = sid * NC + cid

    pltpu.sync_copy(zeros_hbm.at[pl.ds(sid * RPT, RPT)],
                    aggr_sh.at[pl.ds(sid * RPT, RPT)])

    @pl.when(sid == NS - 1)
    def _zero_tail():
        pltpu.sync_copy(zeros_hbm.at[pl.ds(TAIL_OFF, TAIL)],
                        aggr_sh.at[pl.ds(TAIL_OFF, TAIL)])

    plsc.subcore_barrier()

    def mult_chunk(c, b):
        def grp_body(g, c2):
            w16 = ew_a[c, pl.ds(g * 16, 16)]
            for e2 in range(16):
                wv = _lane_splat(w16, e2)
                e = g * 16 + e2
                for j in range(8):
                    rows_v[b, e, pl.ds(j * 16, 16)] = (
                        rows_v[b, e, pl.ds(j * 16, 16)] * wv)
            return c2

        lax.fori_loop(0, KCH // 16, grp_body, 0)

    def drain_scatter(b):
        pltpu.make_async_copy(h_hbm.at[pl.ds(0, KCH)], rows_v.at[b],
                              ssems.at[b]).wait()

    def win_body(s, carry):
        wbase = wid * NCH_W + s * IDXG
        pltpu.sync_copy(edge_hbm.at[0, pl.ds(wbase, IDXG), :], src_a)
        pltpu.sync_copy(edge_hbm.at[1, pl.ds(wbase, IDXG), :], dst_a)
        pltpu.sync_copy(ew_hbm.at[pl.ds(wbase, IDXG), :], ew_a)

        def group_body(g, c2):
            gds = []
            for b in range(NBUF):
                @pl.when(g > 0)
                def _wait_prev():
                    drain_scatter(b)

                gds.append(pltpu.async_copy(
                    h_hbm.at[src_a.at[g * NBUF + b]], rows_v.at[b],
                    gsems.at[b]))
            for b in range(NBUF):
                c = g * NBUF + b
                gds[b].wait()
                mult_chunk(c, b)
                pltpu.async_copy(rows_v.at[b], aggr_sh.at[dst_a.at[c]],
                                 ssems.at[b], add=True)
            return c2

        lax.fori_loop(0, IDXG // NBUF, group_body, 0)
        for b in range(NBUF):
            drain_scatter(b)
        return carry

    lax.fori_loop(0, NWIN, win_body, 0)
    plsc.subcore_barrier()
    pltpu.sync_copy(aggr_sh.at[pl.ds(sid * RPT, RPT)],
                    out_hbm.at[cid, pl.ds(sid * RPT, RPT)])

    @pl.when(sid == NS - 1)
    def _write_tail():
        pltpu.sync_copy(aggr_sh.at[pl.ds(TAIL_OFF, TAIL)],
                        out_hbm.at[cid, pl.ds(TAIL_OFF, TAIL)])


@functools.cache
def _segsum():
    mesh = plsc.VectorSubcoreMesh(core_axis_name="c", subcore_axis_name="s",
                                  num_cores=NC, num_subcores=NS)
    return pl.kernel(
        _segsum_body,
        out_type=jax.ShapeDtypeStruct((NC, N, C), jnp.float32),
        mesh=mesh,
        scratch_types=[
            pltpu.VMEM((IDXG, KCH), jnp.int32),
            pltpu.VMEM((IDXG, KCH), jnp.int32),
            pltpu.VMEM((IDXG, KCH), jnp.float32),
            pltpu.VMEM((NBUF, KCH, C), jnp.float32),
            pltpu.VMEM_SHARED((N, C), jnp.float32),
            pltpu.SemaphoreType.DMA((NBUF,)),
            pltpu.SemaphoreType.DMA((NBUF,)),
        ],
    )


BLK = 2000


def _tc_in_body(x_ref, wt_ref, b_ref, o_ref):
    y = jnp.dot(x_ref[...], wt_ref[...], preferred_element_type=jnp.float32)
    o_ref[...] = jnp.maximum(y + b_ref[...], 0.0)


def _tc_comb_body(p_ref, h_ref, wrelt_ref, wroott_ref, b_ref, g_ref, be_ref,
                  o_ref):
    aggr = p_ref[0] + p_ref[1]
    t = (jnp.dot(aggr, wrelt_ref[...], preferred_element_type=jnp.float32)
         + jnp.dot(h_ref[...], wroott_ref[...],
                   preferred_element_type=jnp.float32)
         + b_ref[...])
    m = jnp.mean(t, axis=-1, keepdims=True)
    v = jnp.mean((t - m) * (t - m), axis=-1, keepdims=True)
    t = (t - m) * lax.rsqrt(v + 1e-5) * g_ref[...] + be_ref[...]
    o_ref[...] = jnp.maximum(t, 0.0)


def _tc_out_body(p_ref, x1_ref, h_ref, wrelt_ref, wroott_ref, b_ref, g_ref,
                 be_ref, woutt_ref, bout_ref, x2_ref, out_ref):
    aggr = p_ref[0] + p_ref[1]
    t = (jnp.dot(aggr, wrelt_ref[...], preferred_element_type=jnp.float32)
         + jnp.dot(x1_ref[...], wroott_ref[...],
                   preferred_element_type=jnp.float32)
         + b_ref[...])
    m = jnp.mean(t, axis=-1, keepdims=True)
    v = jnp.mean((t - m) * (t - m), axis=-1, keepdims=True)
    t = (t - m) * lax.rsqrt(v + 1e-5) * g_ref[...] + be_ref[...]
    x2 = jnp.maximum(t + h_ref[...], 0.0)
    x2_ref[...] = x2
    out_ref[...] = (jnp.dot(x2, woutt_ref[...],
                            preferred_element_type=jnp.float32)
                    + bout_ref[...])


def _row_spec(blk):
    return pl.BlockSpec((blk, C), lambda i: (i, 0))


_W_SPEC = pl.BlockSpec((C, C), lambda i: (0, 0))
_V_SPEC = pl.BlockSpec((1, C), lambda i: (0, 0))
_P_SPEC = pl.BlockSpec((NC, BLK, C), lambda i: (0, i, 0))

_tc_in = pl.pallas_call(
    _tc_in_body,
    grid=(N // BLK,),
    in_specs=[_row_spec(BLK), _W_SPEC, _V_SPEC],
    out_specs=_row_spec(BLK),
    out_shape=jax.ShapeDtypeStruct((N, C), jnp.float32),
)

_tc_comb = pl.pallas_call(
    _tc_comb_body,
    grid=(N // BLK,),
    in_specs=[_P_SPEC, _row_spec(BLK), _W_SPEC, _W_SPEC, _V_SPEC, _V_SPEC,
              _V_SPEC],
    out_specs=_row_spec(BLK),
    out_shape=jax.ShapeDtypeStruct((N, C), jnp.float32),
)

_tc_out = pl.pallas_call(
    _tc_out_body,
    grid=(N // BLK,),
    in_specs=[_P_SPEC, _row_spec(BLK), _row_spec(BLK), _W_SPEC, _W_SPEC,
              _V_SPEC, _V_SPEC, _V_SPEC, _W_SPEC, _V_SPEC],
    out_specs=[_row_spec(BLK), _row_spec(BLK)],
    out_shape=[jax.ShapeDtypeStruct((N, C), jnp.float32),
               jax.ShapeDtypeStruct((N, C), jnp.float32)],
)


def kernel(x, edge, edgeweight, W_in, b_in, Wrel0, brel0, Wroot0, g0, be0,
           Wrel1, brel1, Wroot1, g1, be1, W_out, b_out):
    zeros = jnp.zeros((N, C), jnp.float32)
    pad_n = E_PAD - E
    spread = jnp.arange(pad_n, dtype=jnp.int32) % N
    pad = jnp.stack([spread, spread])
    nchunks = E_PAD // KCH
    perm = (jnp.arange(nchunks, dtype=jnp.int32)
            .reshape(NCH_W, NW).T.reshape(-1))
    edge3 = jnp.concatenate([edge, pad], axis=1).reshape(2, -1, KCH)[:, perm]
    ew2 = jnp.pad(edgeweight, (0, pad_n)).reshape(-1, KCH)[perm]
    h = _tc_in(x, W_in.T, b_in.reshape(1, C))
    p0 = _segsum()(h, edge3, ew2, zeros)
    x1 = _tc_comb(p0, h, Wrel0.T, Wroot0.T, brel0.reshape(1, C),
                  g0.reshape(1, C), be0.reshape(1, C))
    p1 = _segsum()(x1, edge3, ew2, zeros)
    x2, out = _tc_out(p1, x1, h, Wrel1.T, Wroot1.T, brel1.reshape(1, C),
                      g1.reshape(1, C), be1.reshape(1, C), W_out.T,
                      b_out.reshape(1, C))
    return (x2, out)


# speedup vs baseline: 3.2076x; 1.0597x over previous
import functools

import jax
import jax.numpy as jnp
from jax import lax
from jax.experimental import pallas as pl
from jax.experimental.pallas import tpu as pltpu
from jax.experimental.pallas import tpu_sc as plsc

N = 10000
E = 320000
C = 128

NC = 2
NS = 16
NW = NC * NS
KCH = 128
NBUF = 2
IDXG = 16
NCH_W = -(-E // (KCH * NW * IDXG)) * IDXG
E_PAD = NCH_W * NW * KCH
NWIN = NCH_W // IDXG
RPT = 624
TAIL_OFF = RPT * NS
TAIL = N - TAIL_OFF

def _lane_splat(w16, e2):
    return lax.gather(
        w16, jnp.full((16, 1), e2, jnp.int32),
        lax.GatherDimensionNumbers(
            offset_dims=(), collapsed_slice_dims=(0,),
            start_index_map=(0,)),
        (1,), mode=lax.GatherScatterMode.PROMISE_IN_BOUNDS)


def _segsum_body(h_hbm, edge_hbm, ew_hbm, zeros_hbm, out_hbm,
                 src_a, dst_a, ew_a, rows_v, aggr_sh, gsems, ssems):
    cid = lax.axis_index("c")
    sid = lax.axis_index("s")
    wid = sid * NC + cid

    pltpu.sync_copy(zeros_hbm.at[pl.ds(sid * RPT, RPT)],
                    aggr_sh.at[pl.ds(sid * RPT, RPT)])

    @pl.when(sid == NS - 1)
    def _zero_tail():
        pltpu.sync_copy(zeros_hbm.at[pl.ds(TAIL_OFF, TAIL)],
                        aggr_sh.at[pl.ds(TAIL_OFF, TAIL)])

    plsc.subcore_barrier()

    def mult_chunk(c, b):
        def grp_body(g, c2):
            w16 = ew_a[c, pl.ds(g * 16, 16)]
            for e2 in range(16):
                wv = _lane_splat(w16, e2)
                e = g * 16 + e2
                for j in range(8):
                    rows_v[b, e, pl.ds(j * 16, 16)] = (
                        rows_v[b, e, pl.ds(j * 16, 16)] * wv)
            return c2

        lax.fori_loop(0, KCH // 16, grp_body, 0)

    def drain_scatter(b):
        pltpu.make_async_copy(h_hbm.at[pl.ds(0, KCH)], rows_v.at[b],
                              ssems.at[b]).wait()

    def win_body(s, carry):
        wbase = wid * NCH_W + s * IDXG
        pltpu.sync_copy(edge_hbm.at[0, pl.ds(wbase, IDXG), :], src_a)
        pltpu.sync_copy(edge_hbm.at[1, pl.ds(wbase, IDXG), :], dst_a)
        pltpu.sync_copy(ew_hbm.at[pl.ds(wbase, IDXG), :], ew_a)

        def group_body(g, c2):
            gds = []
            for b in range(NBUF):
                @pl.when(g > 0)
                def _wait_prev():
                    drain_scatter(b)

                gds.append(pltpu.async_copy(
                    h_hbm.at[src_a.at[g * NBUF + b]], rows_v.at[b],
                    gsems.at[b]))
            for b in range(NBUF):
                c = g * NBUF + b
                gds[b].wait()
                mult_chunk(c, b)
                pltpu.async_copy(rows_v.at[b], aggr_sh.at[dst_a.at[c]],
                                 ssems.at[b], add=True)
            return c2

        lax.fori_loop(0, IDXG // NBUF, group_body, 0)
        for b in range(NBUF):
            drain_scatter(b)
        return carry

    lax.fori_loop(0, NWIN, win_body, 0)
    plsc.subcore_barrier()
    pltpu.sync_copy(aggr_sh.at[pl.ds(sid * RPT, RPT)],
                    out_hbm.at[cid, pl.ds(sid * RPT, RPT)])

    @pl.when(sid == NS - 1)
    def _write_tail():
        pltpu.sync_copy(aggr_sh.at[pl.ds(TAIL_OFF, TAIL)],
                        out_hbm.at[cid, pl.ds(TAIL_OFF, TAIL)])


@functools.cache
def _segsum():
    mesh = plsc.VectorSubcoreMesh(core_axis_name="c", subcore_axis_name="s",
                                  num_cores=NC, num_subcores=NS)
    return pl.kernel(
        _segsum_body,
        out_type=jax.ShapeDtypeStruct((NC, N, C), jnp.float32),
        mesh=mesh,
        compiler_params=pltpu.CompilerParams(needs_layout_passes=False,
                                             use_tc_tiling_on_sc=False),
        scratch_types=[
            pltpu.VMEM((IDXG, KCH), jnp.int32),
            pltpu.VMEM((IDXG, KCH), jnp.int32),
            pltpu.VMEM((IDXG, KCH), jnp.float32),
            pltpu.VMEM((NBUF, KCH, C), jnp.float32),
            pltpu.VMEM_SHARED((N, C), jnp.float32),
            pltpu.SemaphoreType.DMA((NBUF,)),
            pltpu.SemaphoreType.DMA((NBUF,)),
        ],
    )


BLK = 2000


def _tc_in_body(x_ref, wt_ref, b_ref, o_ref):
    y = jnp.dot(x_ref[...], wt_ref[...], preferred_element_type=jnp.float32)
    o_ref[...] = jnp.maximum(y + b_ref[...], 0.0)


def _tc_comb_body(p_ref, h_ref, wrelt_ref, wroott_ref, b_ref, g_ref, be_ref,
                  o_ref):
    aggr = p_ref[0] + p_ref[1]
    t = (jnp.dot(aggr, wrelt_ref[...], preferred_element_type=jnp.float32)
         + jnp.dot(h_ref[...], wroott_ref[...],
                   preferred_element_type=jnp.float32)
         + b_ref[...])
    m = jnp.mean(t, axis=-1, keepdims=True)
    v = jnp.mean((t - m) * (t - m), axis=-1, keepdims=True)
    t = (t - m) * lax.rsqrt(v + 1e-5) * g_ref[...] + be_ref[...]
    o_ref[...] = jnp.maximum(t, 0.0)


def _tc_out_body(p_ref, x1_ref, h_ref, wrelt_ref, wroott_ref, b_ref, g_ref,
                 be_ref, woutt_ref, bout_ref, x2_ref, out_ref):
    aggr = p_ref[0] + p_ref[1]
    t = (jnp.dot(aggr, wrelt_ref[...], preferred_element_type=jnp.float32)
         + jnp.dot(x1_ref[...], wroott_ref[...],
                   preferred_element_type=jnp.float32)
         + b_ref[...])
    m = jnp.mean(t, axis=-1, keepdims=True)
    v = jnp.mean((t - m) * (t - m), axis=-1, keepdims=True)
    t = (t - m) * lax.rsqrt(v + 1e-5) * g_ref[...] + be_ref[...]
    x2 = jnp.maximum(t + h_ref[...], 0.0)
    x2_ref[...] = x2
    out_ref[...] = (jnp.dot(x2, woutt_ref[...],
                            preferred_element_type=jnp.float32)
                    + bout_ref[...])


def _row_spec(blk):
    return pl.BlockSpec((blk, C), lambda i: (i, 0))


_W_SPEC = pl.BlockSpec((C, C), lambda i: (0, 0))
_V_SPEC = pl.BlockSpec((1, C), lambda i: (0, 0))
_P_SPEC = pl.BlockSpec((NC, BLK, C), lambda i: (0, i, 0))

_tc_in = pl.pallas_call(
    _tc_in_body,
    grid=(N // BLK,),
    in_specs=[_row_spec(BLK), _W_SPEC, _V_SPEC],
    out_specs=_row_spec(BLK),
    out_shape=jax.ShapeDtypeStruct((N, C), jnp.float32),
)

_tc_comb = pl.pallas_call(
    _tc_comb_body,
    grid=(N // BLK,),
    in_specs=[_P_SPEC, _row_spec(BLK), _W_SPEC, _W_SPEC, _V_SPEC, _V_SPEC,
              _V_SPEC],
    out_specs=_row_spec(BLK),
    out_shape=jax.ShapeDtypeStruct((N, C), jnp.float32),
)

_tc_out = pl.pallas_call(
    _tc_out_body,
    grid=(N // BLK,),
    in_specs=[_P_SPEC, _row_spec(BLK), _row_spec(BLK), _W_SPEC, _W_SPEC,
              _V_SPEC, _V_SPEC, _V_SPEC, _W_SPEC, _V_SPEC],
    out_specs=[_row_spec(BLK), _row_spec(BLK)],
    out_shape=[jax.ShapeDtypeStruct((N, C), jnp.float32),
               jax.ShapeDtypeStruct((N, C), jnp.float32)],
)


def kernel(x, edge, edgeweight, W_in, b_in, Wrel0, brel0, Wroot0, g0, be0,
           Wrel1, brel1, Wroot1, g1, be1, W_out, b_out):
    zeros = jnp.zeros((N, C), jnp.float32)
    pad_n = E_PAD - E
    spread = jnp.arange(pad_n, dtype=jnp.int32) % N
    pad = jnp.stack([spread, spread])
    nchunks = E_PAD // KCH
    perm = (jnp.arange(nchunks, dtype=jnp.int32)
            .reshape(NCH_W, NW).T.reshape(-1))
    edge3 = jnp.concatenate([edge, pad], axis=1).reshape(2, -1, KCH)[:, perm]
    ew2 = jnp.pad(edgeweight, (0, pad_n)).reshape(-1, KCH)[perm]
    h = _tc_in(x, W_in.T, b_in.reshape(1, C))
    p0 = _segsum()(h, edge3, ew2, zeros)
    x1 = _tc_comb(p0, h, Wrel0.T, Wroot0.T, brel0.reshape(1, C),
                  g0.reshape(1, C), be0.reshape(1, C))
    p1 = _segsum()(x1, edge3, ew2, zeros)
    x2, out = _tc_out(p1, x1, h, Wrel1.T, Wroot1.T, brel1.reshape(1, C),
                      g1.reshape(1, C), be1.reshape(1, C), W_out.T,
                      b_out.reshape(1, C))
    return (x2, out)


# speedup vs baseline: 3.3256x; 1.0368x over previous
import functools

import jax
import jax.numpy as jnp
from jax import lax
from jax.experimental import pallas as pl
from jax.experimental.pallas import tpu as pltpu
from jax.experimental.pallas import tpu_sc as plsc

N = 10000
E = 320000
C = 128

NC = 2
NS = 16
NW = NC * NS
KCH = 128
NBUF = 2
IDXG = 40
NCH_W = -(-E // (KCH * NW * IDXG)) * IDXG
E_PAD = NCH_W * NW * KCH
NWIN = NCH_W // IDXG
RPT = 624
TAIL_OFF = RPT * NS
TAIL = N - TAIL_OFF

def _lane_splat(w16, e2):
    return lax.gather(
        w16, jnp.full((16, 1), e2, jnp.int32),
        lax.GatherDimensionNumbers(
            offset_dims=(), collapsed_slice_dims=(0,),
            start_index_map=(0,)),
        (1,), mode=lax.GatherScatterMode.PROMISE_IN_BOUNDS)


def _segsum_body(h_hbm, edge_hbm, ew_hbm, zeros_hbm, out_hbm,
                 src_a, dst_a, ew_a, rows_v, aggr_sh, gsems, ssems):
    cid = lax.axis_index("c")
    sid = lax.axis_index("s")
    wid = sid * NC + cid

    pltpu.sync_copy(zeros_hbm.at[pl.ds(sid * RPT, RPT)],
                    aggr_sh.at[pl.ds(sid * RPT, RPT)])

    @pl.when(sid == NS - 1)
    def _zero_tail():
        pltpu.sync_copy(zeros_hbm.at[pl.ds(TAIL_OFF, TAIL)],
                        aggr_sh.at[pl.ds(TAIL_OFF, TAIL)])

    plsc.subcore_barrier()

    def mult_chunk(c, b):
        def grp_body(g, c2):
            w16 = ew_a[c, pl.ds(g * 16, 16)]
            for e2 in range(16):
                wv = _lane_splat(w16, e2)
                e = g * 16 + e2
                for j in range(8):
                    rows_v[b, e, pl.ds(j * 16, 16)] = (
                        rows_v[b, e, pl.ds(j * 16, 16)] * wv)
            return c2

        lax.fori_loop(0, KCH // 16, grp_body, 0)

    def drain_scatter(b):
        pltpu.make_async_copy(h_hbm.at[pl.ds(0, KCH)], rows_v.at[b],
                              ssems.at[b]).wait()

    def win_body(s, carry):
        wbase = wid * NCH_W + s * IDXG
        pltpu.sync_copy(edge_hbm.at[0, pl.ds(wbase, IDXG), :], src_a)
        pltpu.sync_copy(edge_hbm.at[1, pl.ds(wbase, IDXG), :], dst_a)
        pltpu.sync_copy(ew_hbm.at[pl.ds(wbase, IDXG), :], ew_a)

        def group_body(g, c2):
            gds = []
            for b in range(NBUF):
                @pl.when(g > 0)
                def _wait_prev():
                    drain_scatter(b)

                gds.append(pltpu.async_copy(
                    h_hbm.at[src_a.at[g * NBUF + b]], rows_v.at[b],
                    gsems.at[b]))
            for b in range(NBUF):
                c = g * NBUF + b
                gds[b].wait()
                mult_chunk(c, b)
                pltpu.async_copy(rows_v.at[b], aggr_sh.at[dst_a.at[c]],
                                 ssems.at[b], add=True)
            return c2

        lax.fori_loop(0, IDXG // NBUF, group_body, 0)
        for b in range(NBUF):
            drain_scatter(b)
        return carry

    lax.fori_loop(0, NWIN, win_body, 0)
    plsc.subcore_barrier()
    pltpu.sync_copy(aggr_sh.at[pl.ds(sid * RPT, RPT)],
                    out_hbm.at[cid, pl.ds(sid * RPT, RPT)])

    @pl.when(sid == NS - 1)
    def _write_tail():
        pltpu.sync_copy(aggr_sh.at[pl.ds(TAIL_OFF, TAIL)],
                        out_hbm.at[cid, pl.ds(TAIL_OFF, TAIL)])


@functools.cache
def _segsum():
    mesh = plsc.VectorSubcoreMesh(core_axis_name="c", subcore_axis_name="s",
                                  num_cores=NC, num_subcores=NS)
    return pl.kernel(
        _segsum_body,
        out_type=jax.ShapeDtypeStruct((NC, N, C), jnp.float32),
        mesh=mesh,
        compiler_params=pltpu.CompilerParams(needs_layout_passes=False,
                                             use_tc_tiling_on_sc=False),
        scratch_types=[
            pltpu.VMEM((IDXG, KCH), jnp.int32),
            pltpu.VMEM((IDXG, KCH), jnp.int32),
            pltpu.VMEM((IDXG, KCH), jnp.float32),
            pltpu.VMEM((NBUF, KCH, C), jnp.float32),
            pltpu.VMEM_SHARED((N, C), jnp.float32),
            pltpu.SemaphoreType.DMA((NBUF,)),
            pltpu.SemaphoreType.DMA((NBUF,)),
        ],
    )


BLK = 2000


def _tc_in_body(x_ref, wt_ref, b_ref, o_ref):
    y = jnp.dot(x_ref[...], wt_ref[...], preferred_element_type=jnp.float32)
    o_ref[...] = jnp.maximum(y + b_ref[...], 0.0)


def _tc_comb_body(p_ref, h_ref, wrelt_ref, wroott_ref, b_ref, g_ref, be_ref,
                  o_ref):
    aggr = p_ref[0] + p_ref[1]
    t = (jnp.dot(aggr, wrelt_ref[...], preferred_element_type=jnp.float32)
         + jnp.dot(h_ref[...], wroott_ref[...],
                   preferred_element_type=jnp.float32)
         + b_ref[...])
    m = jnp.mean(t, axis=-1, keepdims=True)
    v = jnp.mean((t - m) * (t - m), axis=-1, keepdims=True)
    t = (t - m) * lax.rsqrt(v + 1e-5) * g_ref[...] + be_ref[...]
    o_ref[...] = jnp.maximum(t, 0.0)


def _tc_out_body(p_ref, x1_ref, h_ref, wrelt_ref, wroott_ref, b_ref, g_ref,
                 be_ref, woutt_ref, bout_ref, x2_ref, out_ref):
    aggr = p_ref[0] + p_ref[1]
    t = (jnp.dot(aggr, wrelt_ref[...], preferred_element_type=jnp.float32)
         + jnp.dot(x1_ref[...], wroott_ref[...],
                   preferred_element_type=jnp.float32)
         + b_ref[...])
    m = jnp.mean(t, axis=-1, keepdims=True)
    v = jnp.mean((t - m) * (t - m), axis=-1, keepdims=True)
    t = (t - m) * lax.rsqrt(v + 1e-5) * g_ref[...] + be_ref[...]
    x2 = jnp.maximum(t + h_ref[...], 0.0)
    x2_ref[...] = x2
    out_ref[...] = (jnp.dot(x2, woutt_ref[...],
                            preferred_element_type=jnp.float32)
                    + bout_ref[...])


def _row_spec(blk):
    return pl.BlockSpec((blk, C), lambda i: (i, 0))


_W_SPEC = pl.BlockSpec((C, C), lambda i: (0, 0))
_V_SPEC = pl.BlockSpec((1, C), lambda i: (0, 0))
_P_SPEC = pl.BlockSpec((NC, BLK, C), lambda i: (0, i, 0))

_tc_in = pl.pallas_call(
    _tc_in_body,
    grid=(N // BLK,),
    in_specs=[_row_spec(BLK), _W_SPEC, _V_SPEC],
    out_specs=_row_spec(BLK),
    out_shape=jax.ShapeDtypeStruct((N, C), jnp.float32),
)

_tc_comb = pl.pallas_call(
    _tc_comb_body,
    grid=(N // BLK,),
    in_specs=[_P_SPEC, _row_spec(BLK), _W_SPEC, _W_SPEC, _V_SPEC, _V_SPEC,
              _V_SPEC],
    out_specs=_row_spec(BLK),
    out_shape=jax.ShapeDtypeStruct((N, C), jnp.float32),
)

_tc_out = pl.pallas_call(
    _tc_out_body,
    grid=(N // BLK,),
    in_specs=[_P_SPEC, _row_spec(BLK), _row_spec(BLK), _W_SPEC, _W_SPEC,
              _V_SPEC, _V_SPEC, _V_SPEC, _W_SPEC, _V_SPEC],
    out_specs=[_row_spec(BLK), _row_spec(BLK)],
    out_shape=[jax.ShapeDtypeStruct((N, C), jnp.float32),
               jax.ShapeDtypeStruct((N, C), jnp.float32)],
)


def kernel(x, edge, edgeweight, W_in, b_in, Wrel0, brel0, Wroot0, g0, be0,
           Wrel1, brel1, Wroot1, g1, be1, W_out, b_out):
    zeros = jnp.zeros((N, C), jnp.float32)
    pad_n = E_PAD - E
    spread = jnp.arange(pad_n, dtype=jnp.int32) % N
    pad = jnp.stack([spread, spread])
    nchunks = E_PAD // KCH
    perm = (jnp.arange(nchunks, dtype=jnp.int32)
            .reshape(NCH_W, NW).T.reshape(-1))
    edge3 = jnp.concatenate([edge, pad], axis=1).reshape(2, -1, KCH)[:, perm]
    ew2 = jnp.pad(edgeweight, (0, pad_n)).reshape(-1, KCH)[perm]
    h = _tc_in(x, W_in.T, b_in.reshape(1, C))
    p0 = _segsum()(h, edge3, ew2, zeros)
    x1 = _tc_comb(p0, h, Wrel0.T, Wroot0.T, brel0.reshape(1, C),
                  g0.reshape(1, C), be0.reshape(1, C))
    p1 = _segsum()(x1, edge3, ew2, zeros)
    x2, out = _tc_out(p1, x1, h, Wrel1.T, Wroot1.T, brel1.reshape(1, C),
                      g1.reshape(1, C), be1.reshape(1, C), W_out.T,
                      b_out.reshape(1, C))
    return (x2, out)
